# baseline (speedup 1.0000x reference)
import jax
import jax.numpy as jnp
from jax import lax
from jax.experimental import pallas as pl
from jax.experimental.pallas import tpu as pltpu
from jax.experimental.pallas import tpu_sc as plsc

B, F, D, NI, NU = 16384, 256, 128, 1000, 16384
NFLAT = NI * NU
BB = 1024
NBLK = B // BB

NW = 32
REG = NFLAT // NW
ZCH = 16000
NZ = REG // ZCH
NRING = 3


def _ratings_kernel(idx_ref, uf_ref, wu_ref, bu_ref, tab_ref, waff_ref,
                    baff_ref, out_ref, logits_ref, acc_ref):
    i = pl.program_id(0)

    @pl.when(i == 0)
    def _init():
        acc_ref[0] = 0.0
        acc_ref[1] = 0.0

    @pl.when(i < NBLK)
    def _compute():
        idx = idx_ref[0, 0, :]
        onehot = (idx[:, None] ==
                  lax.broadcasted_iota(jnp.int32, (BB, NI), 1)
                  ).astype(jnp.bfloat16)
        g = jnp.dot(onehot, tab_ref[...].astype(jnp.bfloat16),
                    preferred_element_type=jnp.float32)
        a = jnp.dot(uf_ref[...].astype(jnp.bfloat16),
                    wu_ref[...].astype(jnp.bfloat16),
                    preferred_element_type=jnp.float32) + bu_ref[...]
        prod = a * g
        l = jnp.sum(prod * waff_ref[...], axis=1) + baff_ref[0, 0]
        logits_ref[pl.ds(i, 1), :] = l[None, :]
        acc_ref[0] += jnp.sum(l)
        acc_ref[1] += jnp.sum(l * l)

    @pl.when(i == NBLK)
    def _finalize():
        s = acc_ref[0]
        q = acc_ref[1]
        mean = s / B
        var = (q - s * s / B) / (B - 1)
        rstd = 1.0 / (jnp.sqrt(var) + 1e-5)
        z = (logits_ref[...] - mean) * rstd
        out_ref[...] = 1.0 / (1.0 + jnp.exp(-z))


def _compute_ratings(item_idx, uf, wu, bu, tab, waff, baff):
    idx3 = item_idx.reshape(NBLK, 1, BB)
    bu2 = bu.reshape(1, D)
    waff2 = waff.reshape(1, D)
    baff2 = baff.reshape(1, 1)
    last = NBLK - 1
    return pl.pallas_call(
        _ratings_kernel,
        grid=(NBLK + 1,),
        in_specs=[
            pl.BlockSpec((1, 1, BB), lambda i: (jnp.minimum(i, last), 0, 0)),
            pl.BlockSpec((BB, F), lambda i: (jnp.minimum(i, last), 0)),
            pl.BlockSpec((F, D), lambda i: (0, 0)),
            pl.BlockSpec((1, D), lambda i: (0, 0)),
            pl.BlockSpec((NI, D), lambda i: (0, 0)),
            pl.BlockSpec((1, D), lambda i: (0, 0)),
            pl.BlockSpec((1, 1), lambda i: (0, 0)),
        ],
        out_specs=pl.BlockSpec((NBLK, BB), lambda i: (0, 0)),
        out_shape=jax.ShapeDtypeStruct((NBLK, BB), jnp.float32),
        scratch_shapes=[
            pltpu.VMEM((NBLK, BB), jnp.float32),
            pltpu.SMEM((2,), jnp.float32),
        ],
    )(idx3, uf, wu, bu2, tab, waff2, baff2).reshape(B)


def _sc_scatter_body(soff_hbm, perm_hbm, rat_hbm, bnd_hbm, out_hbm,
                     zbuf, offv, permv, ratv, bndv, zsem):
    wid = lax.axis_index("s") * 2 + lax.axis_index("c")
    base = wid * REG

    pltpu.sync_copy(soff_hbm, offv.at[pl.ds(0, B)])
    pltpu.sync_copy(perm_hbm, permv.at[pl.ds(0, B)])
    pltpu.sync_copy(rat_hbm, ratv)
    pltpu.sync_copy(bnd_hbm, bndv)

    bpair = bndv[pl.ds(wid, 16)]
    start = bpair[0]
    end = bpair[1]

    one16 = jnp.full((16,), 1, jnp.int32)
    zero16i = jnp.zeros((16,), jnp.int32)
    lanes = lax.iota(jnp.int32, 16)
    basev = jnp.full((16,), base, jnp.int32)
    startv = jnp.full((16,), start, jnp.int32)
    endv = jnp.full((16,), end, jnp.int32)

    def _compact(j, cnt):
        o = offv[pl.ds(j * 16, 16)]
        gidx = lanes + jnp.full((16,), j * 16, jnp.int32)
        m = (gidx >= startv) & (gidx < endv) & (o >= zero16i)
        p = permv[pl.ds(j * 16, 16)]
        v = plsc.load_gather(ratv, [p])
        ps = plsc.cumsum(jnp.where(m, one16, zero16i))
        pos = (ps - one16) + jnp.full((16,), cnt, jnp.int32)
        plsc.store_scatter(offv, [pos], o - basev, mask=m)
        plsc.store_scatter(permv, [pos], plsc.bitcast(v, jnp.int32), mask=m)
        return cnt + ps[15]

    cnt = lax.fori_loop(start // 16, (end + 15) // 16, _compact,
                        jnp.int32(0))
    nv = (cnt + 15) // 16

    @pl.when(cnt > 0)
    def _pad():
        lastv_o = offv[pl.ds(cnt - 1, 16)][0]
        lastv_p = permv[pl.ds(cnt - 1, 16)][0]
        j = (cnt - 1) // 16
        m = lanes < jnp.full((16,), cnt - j * 16, jnp.int32)
        offv[pl.ds(j * 16, 16)] = jnp.where(
            m, offv[pl.ds(j * 16, 16)], jnp.full((16,), lastv_o, jnp.int32))
        permv[pl.ds(j * 16, 16)] = jnp.where(
            m, permv[pl.ds(j * 16, 16)], jnp.full((16,), lastv_p, jnp.int32))

    zero16 = jnp.zeros((16,), jnp.float32)

    def _emit(k, carry):
        boff = (k % NRING) * ZCH

        @pl.when(k >= NRING)
        def _drain_prev():
            pltpu.make_async_copy(
                zbuf.at[pl.ds(0, ZCH)], out_hbm.at[pl.ds(base, ZCH)],
                zsem).wait()

        def _memset(j, c2):
            zbuf[pl.ds(boff + j * 16, 16)] = zero16
            return c2

        lax.fori_loop(0, ZCH // 16, _memset, 0)

        wlo = jnp.full((16,), k * ZCH - boff, jnp.int32)
        whi = jnp.full((16,), k * ZCH, jnp.int32)
        wh2 = jnp.full((16,), (k + 1) * ZCH, jnp.int32)

        def _insert(j, c2):
            rel = offv[pl.ds(j * 16, 16)]
            m = (rel >= whi) & (rel < wh2)
            val = plsc.bitcast(permv[pl.ds(j * 16, 16)], jnp.float32)
            plsc.store_scatter(zbuf, [rel - wlo], val, mask=m)
            return c2

        lax.fori_loop(0, nv, _insert, 0)

        pltpu.async_copy(zbuf.at[pl.ds(boff, ZCH)],
                         out_hbm.at[pl.ds(base + k * ZCH, ZCH)], zsem)
        return carry

    lax.fori_loop(0, NZ, _emit, 0)

    def _drain(k, carry):
        pltpu.make_async_copy(
            zbuf.at[pl.ds(0, ZCH)], out_hbm.at[pl.ds(base, ZCH)],
            zsem).wait()
        return carry

    lax.fori_loop(0, NRING, _drain, 0)


def _sc_scatter(soff_sorted, perm, ratings, bounds):
    mesh = plsc.VectorSubcoreMesh(core_axis_name="c", subcore_axis_name="s",
                                  num_cores=2, num_subcores=16)
    kern = pl.kernel(
        _sc_scatter_body,
        out_type=jax.ShapeDtypeStruct((NFLAT,), jnp.float32),
        mesh=mesh,
        compiler_params=pltpu.CompilerParams(needs_layout_passes=False),
        scratch_types=[
            pltpu.VMEM((NRING * ZCH,), jnp.float32),
            pltpu.VMEM((B + 16,), jnp.int32),
            pltpu.VMEM((B + 16,), jnp.int32),
            pltpu.VMEM((B,), jnp.float32),
            pltpu.VMEM((NW + 16,), jnp.int32),
            pltpu.SemaphoreType.DMA,
        ],
    )
    return kern(soff_sorted, perm, ratings, bounds)


def kernel(user_features, item_indices, user_indices, W_user, b_user,
           item_table, W_aff, b_aff):
    ii = item_indices.astype(jnp.int32)
    ui = user_indices.astype(jnp.int32)
    flat = ii * NU + ui

    iotaf = lax.iota(jnp.float32, B)
    ks, vs = lax.sort((flat, iotaf), num_keys=1, is_stable=False)
    is_last = jnp.concatenate(
        [ks[1:] != ks[:-1], jnp.ones((1,), jnp.bool_)])
    soff_sorted = jnp.where(is_last, ks, -1)
    perm = vs.astype(jnp.int32)
    bounds = jnp.searchsorted(
        ks, jnp.arange(NW + 1, dtype=jnp.int32) * REG).astype(jnp.int32)
    bounds = jnp.concatenate(
        [bounds, jnp.zeros((15,), jnp.int32)])

    ratings = _compute_ratings(ii, user_features, W_user, b_user,
                               item_table, W_aff, b_aff)
    c = _sc_scatter(soff_sorted, perm, ratings, bounds)
    return c.reshape(1, NI, NU)


# speedup vs baseline: 1.5634x; 1.5634x over previous
import jax
import jax.numpy as jnp
from jax import lax
from jax.experimental import pallas as pl
from jax.experimental.pallas import tpu as pltpu
from jax.experimental.pallas import tpu_sc as plsc

B, F, D, NI, NU = 16384, 256, 128, 1000, 16384
NFLAT = NI * NU
BB = 1024
NBLK = B // BB

NW = 32
REG = NFLAT // NW
ZCH = 16000
NZ = REG // ZCH
NRING = 3


def _ratings_kernel(idx_ref, uf_ref, wu_ref, bu_ref, tab_ref, waff_ref,
                    baff_ref, out_ref, logits_ref, acc_ref):
    i = pl.program_id(0)

    @pl.when(i == 0)
    def _init():
        acc_ref[0] = 0.0
        acc_ref[1] = 0.0

    @pl.when(i < NBLK)
    def _compute():
        idx = idx_ref[0, 0, :]
        onehot = (idx[:, None] ==
                  lax.broadcasted_iota(jnp.int32, (BB, NI), 1)
                  ).astype(jnp.bfloat16)
        g = jnp.dot(onehot, tab_ref[...].astype(jnp.bfloat16),
                    preferred_element_type=jnp.float32)
        a = jnp.dot(uf_ref[...].astype(jnp.bfloat16),
                    wu_ref[...].astype(jnp.bfloat16),
                    preferred_element_type=jnp.float32) + bu_ref[...]
        prod = a * g
        l = jnp.sum(prod * waff_ref[...], axis=1) + baff_ref[0, 0]
        logits_ref[pl.ds(i, 1), :] = l[None, :]
        acc_ref[0] += jnp.sum(l)
        acc_ref[1] += jnp.sum(l * l)

    @pl.when(i == NBLK)
    def _finalize():
        s = acc_ref[0]
        q = acc_ref[1]
        mean = s / B
        var = (q - s * s / B) / (B - 1)
        rstd = 1.0 / (jnp.sqrt(var) + 1e-5)
        z = (logits_ref[...] - mean) * rstd
        out_ref[...] = 1.0 / (1.0 + jnp.exp(-z))


def _compute_ratings(item_idx, uf, wu, bu, tab, waff, baff):
    idx3 = item_idx.reshape(NBLK, 1, BB)
    bu2 = bu.reshape(1, D)
    waff2 = waff.reshape(1, D)
    baff2 = baff.reshape(1, 1)
    last = NBLK - 1
    return pl.pallas_call(
        _ratings_kernel,
        grid=(NBLK + 1,),
        in_specs=[
            pl.BlockSpec((1, 1, BB), lambda i: (jnp.minimum(i, last), 0, 0)),
            pl.BlockSpec((BB, F), lambda i: (jnp.minimum(i, last), 0)),
            pl.BlockSpec((F, D), lambda i: (0, 0)),
            pl.BlockSpec((1, D), lambda i: (0, 0)),
            pl.BlockSpec((NI, D), lambda i: (0, 0)),
            pl.BlockSpec((1, D), lambda i: (0, 0)),
            pl.BlockSpec((1, 1), lambda i: (0, 0)),
        ],
        out_specs=pl.BlockSpec((NBLK, BB), lambda i: (0, 0)),
        out_shape=jax.ShapeDtypeStruct((NBLK, BB), jnp.float32),
        scratch_shapes=[
            pltpu.VMEM((NBLK, BB), jnp.float32),
            pltpu.SMEM((2,), jnp.float32),
        ],
    )(idx3, uf, wu, bu2, tab, waff2, baff2).reshape(B)


def _sc_scatter_body(soff_hbm, perm_hbm, rat_hbm, bnd_hbm, out_hbm,
                     zbuf, offv, permv, ratv, bndv, zsem):
    wid = lax.axis_index("s") * 2 + lax.axis_index("c")
    base = wid * REG

    pltpu.sync_copy(soff_hbm, offv.at[pl.ds(0, B)])
    pltpu.sync_copy(perm_hbm, permv.at[pl.ds(0, B)])
    pltpu.sync_copy(rat_hbm, ratv)
    pltpu.sync_copy(bnd_hbm, bndv)

    bpair = bndv[pl.ds(wid, 16)]
    start = bpair[0]
    end = bpair[1]

    one16 = jnp.full((16,), 1, jnp.int32)
    zero16i = jnp.zeros((16,), jnp.int32)
    lanes = lax.iota(jnp.int32, 16)
    basev = jnp.full((16,), base, jnp.int32)
    startv = jnp.full((16,), start, jnp.int32)
    endv = jnp.full((16,), end, jnp.int32)

    def _compact(j, cnt):
        o = offv[pl.ds(j * 16, 16)]
        gidx = lanes + jnp.full((16,), j * 16, jnp.int32)
        m = (gidx >= startv) & (gidx < endv) & (o >= zero16i)
        p = permv[pl.ds(j * 16, 16)]
        v = plsc.load_gather(ratv, [p])
        ps = plsc.cumsum(jnp.where(m, one16, zero16i))
        pos = (ps - one16) + jnp.full((16,), cnt, jnp.int32)
        plsc.store_scatter(offv, [pos], o - basev, mask=m)
        plsc.store_scatter(permv, [pos], plsc.bitcast(v, jnp.int32), mask=m)
        return cnt + ps[15]

    cnt = lax.fori_loop(start // 16, (end + 15) // 16, _compact,
                        jnp.int32(0))
    nv = (cnt + 15) // 16

    @pl.when(cnt > 0)
    def _pad():
        lastv_o = offv[pl.ds(cnt - 1, 16)][0]
        lastv_p = permv[pl.ds(cnt - 1, 16)][0]
        j = (cnt - 1) // 16
        m = lanes < jnp.full((16,), cnt - j * 16, jnp.int32)
        offv[pl.ds(j * 16, 16)] = jnp.where(
            m, offv[pl.ds(j * 16, 16)], jnp.full((16,), lastv_o, jnp.int32))
        permv[pl.ds(j * 16, 16)] = jnp.where(
            m, permv[pl.ds(j * 16, 16)], jnp.full((16,), lastv_p, jnp.int32))

    zero16 = jnp.zeros((16,), jnp.float32)

    def _memset(j, c2):
        for t in range(16):
            zbuf[pl.ds(j * 256 + t * 16, 16)] = zero16
        return c2

    lax.fori_loop(0, (NRING * ZCH) // 256, _memset, 0)

    def _emit(k, carry):
        boff = (k % NRING) * ZCH
        boffv = jnp.full((16,), boff, jnp.int32)

        @pl.when(k >= NRING)
        def _clean_prev():
            pltpu.make_async_copy(
                zbuf.at[pl.ds(0, ZCH)], out_hbm.at[pl.ds(base, ZCH)],
                zsem).wait()
            plo = jnp.full((16,), (k - NRING) * ZCH, jnp.int32)
            phi = jnp.full((16,), (k - NRING + 1) * ZCH, jnp.int32)

            def _unzero(j, c2):
                rel = offv[pl.ds(j * 16, 16)]
                m = (rel >= plo) & (rel < phi)
                plsc.store_scatter(zbuf, [(rel - plo) + boffv], zero16,
                                   mask=m)
                return c2

            lax.fori_loop(0, nv, _unzero, 0)

        wlo = jnp.full((16,), k * ZCH, jnp.int32)
        whi = jnp.full((16,), (k + 1) * ZCH, jnp.int32)

        def _insert(j, c2):
            rel = offv[pl.ds(j * 16, 16)]
            m = (rel >= wlo) & (rel < whi)
            val = plsc.bitcast(permv[pl.ds(j * 16, 16)], jnp.float32)
            plsc.store_scatter(zbuf, [(rel - wlo) + boffv], val, mask=m)
            return c2

        lax.fori_loop(0, nv, _insert, 0)

        pltpu.async_copy(zbuf.at[pl.ds(boff, ZCH)],
                         out_hbm.at[pl.ds(base + k * ZCH, ZCH)], zsem)
        return carry

    lax.fori_loop(0, NZ, _emit, 0)

    def _drain(k, carry):
        pltpu.make_async_copy(
            zbuf.at[pl.ds(0, ZCH)], out_hbm.at[pl.ds(base, ZCH)],
            zsem).wait()
        return carry

    lax.fori_loop(0, NRING, _drain, 0)


def _sc_scatter(soff_sorted, perm, ratings, bounds):
    mesh = plsc.VectorSubcoreMesh(core_axis_name="c", subcore_axis_name="s",
                                  num_cores=2, num_subcores=16)
    kern = pl.kernel(
        _sc_scatter_body,
        out_type=jax.ShapeDtypeStruct((NFLAT,), jnp.float32),
        mesh=mesh,
        compiler_params=pltpu.CompilerParams(needs_layout_passes=False),
        scratch_types=[
            pltpu.VMEM((NRING * ZCH,), jnp.float32),
            pltpu.VMEM((B + 16,), jnp.int32),
            pltpu.VMEM((B + 16,), jnp.int32),
            pltpu.VMEM((B,), jnp.float32),
            pltpu.VMEM((NW + 16,), jnp.int32),
            pltpu.SemaphoreType.DMA,
        ],
    )
    return kern(soff_sorted, perm, ratings, bounds)


def kernel(user_features, item_indices, user_indices, W_user, b_user,
           item_table, W_aff, b_aff):
    ii = item_indices.astype(jnp.int32)
    ui = user_indices.astype(jnp.int32)
    flat = ii * NU + ui

    iotaf = lax.iota(jnp.float32, B)
    ks, vs = lax.sort((flat, iotaf), num_keys=1, is_stable=False)
    is_last = jnp.concatenate(
        [ks[1:] != ks[:-1], jnp.ones((1,), jnp.bool_)])
    soff_sorted = jnp.where(is_last, ks, -1)
    perm = vs.astype(jnp.int32)
    bounds = jnp.searchsorted(
        ks, jnp.arange(NW + 1, dtype=jnp.int32) * REG).astype(jnp.int32)
    bounds = jnp.concatenate(
        [bounds, jnp.zeros((15,), jnp.int32)])

    ratings = _compute_ratings(ii, user_features, W_user, b_user,
                               item_table, W_aff, b_aff)
    c = _sc_scatter(soff_sorted, perm, ratings, bounds)
    return c.reshape(1, NI, NU)
